# baseline (speedup 1.0000x reference)
import functools

import jax
import jax.numpy as jnp
from jax.experimental import pallas as pl
from jax.experimental.pallas import tpu as pltpu

_CK = 1024
_NFULL = 48
_TAIL = 848
_RA = 5
_RB = 5
_RO = 6


def _rag_kernel(q_hbm, wq_hbm, bq_ref, w1_ref, b1_ref, ce_ref, b2_ref,
                w2_hbm, out_hbm,
                qbuf, wqbuf, qtail, wqtail, w2buf, w2tail, outbuf, outtail,
                acc_ref, h_ref,
                sem_q, sem_wq, sem_qt, sem_wqt, sem_w2, sem_w2t, sem_out):
    batch = q_hbm.shape[0]
    embed = wq_hbm.shape[0]
    hidden = w1_ref.shape[0]

    def a_copies(k, s):
        off = k * _CK
        return (
            pltpu.make_async_copy(q_hbm.at[:, pl.ds(off, _CK)],
                                  qbuf.at[s], sem_q.at[s]),
            pltpu.make_async_copy(wq_hbm.at[:, pl.ds(off, _CK)],
                                  wqbuf.at[s], sem_wq.at[s]),
        )

    for k in range(_RA):
        for c in a_copies(k, k):
            c.start()
    pltpu.make_async_copy(q_hbm.at[:, pl.ds(_NFULL * _CK, _TAIL)],
                          qtail, sem_qt).start()
    pltpu.make_async_copy(wq_hbm.at[:, pl.ds(_NFULL * _CK, _TAIL)],
                          wqtail, sem_wqt).start()

    acc_ref[...] = jnp.zeros_like(acc_ref)

    def a_body(k, carry):
        s = jax.lax.rem(k, _RA)
        cq, cwq = a_copies(k, s)
        cq.wait()
        cwq.wait()
        qb = qbuf[s].astype(jnp.bfloat16)
        wb = wqbuf[s].astype(jnp.bfloat16)
        acc_ref[...] += jax.lax.dot_general(
            qb, wb, (((1,), (1,)), ((), ())),
            preferred_element_type=jnp.float32)

        @pl.when(k + _RA < _NFULL)
        def _issue_next():
            for c in a_copies(k + _RA, s):
                c.start()
        return carry

    jax.lax.fori_loop(0, _NFULL, a_body, None)

    pltpu.make_async_copy(q_hbm.at[:, pl.ds(_NFULL * _CK, _TAIL)],
                          qtail, sem_qt).wait()
    pltpu.make_async_copy(wq_hbm.at[:, pl.ds(_NFULL * _CK, _TAIL)],
                          wqtail, sem_wqt).wait()
    acc_ref[...] += jax.lax.dot_general(
        qtail[...].astype(jnp.bfloat16), wqtail[...].astype(jnp.bfloat16),
        (((1,), (1,)), ((), ())), preferred_element_type=jnp.float32)

    def b_copy(k, s):
        return pltpu.make_async_copy(w2_hbm.at[pl.ds(k * _CK, _CK), :],
                                     w2buf.at[s], sem_w2.at[s])

    for k in range(_RB):
        b_copy(k, k).start()
    pltpu.make_async_copy(w2_hbm.at[pl.ds(_NFULL * _CK, _TAIL), :],
                          w2tail, sem_w2t).start()

    qe = (acc_ref[...] + bq_ref[...]).astype(jnp.bfloat16)
    ce = ce_ref[...].astype(jnp.bfloat16)
    w1 = w1_ref[...]
    w1a = w1[:, :embed].astype(jnp.bfloat16)
    w1b = w1[:, embed:].astype(jnp.bfloat16)
    pre = jax.lax.dot_general(qe, w1a, (((1,), (1,)), ((), ())),
                              preferred_element_type=jnp.float32)
    pre += jax.lax.dot_general(ce, w1b, (((1,), (1,)), ((), ())),
                               preferred_element_type=jnp.float32)
    pre += b1_ref[...]
    h_ref[...] = jnp.maximum(pre, 0.0).astype(jnp.bfloat16)

    def out_copy(k, so):
        return pltpu.make_async_copy(
            outbuf.at[so], out_hbm.at[:, pl.ds(k * _CK, _CK)],
            sem_out.at[so])

    def b_body(k, carry):
        s = jax.lax.rem(k, _RB)
        so = jax.lax.rem(k, _RO)
        b_copy(k, s).wait()
        wb = w2buf[s].astype(jnp.bfloat16)
        res = jax.lax.dot_general(
            h_ref[...], wb, (((1,), (1,)), ((), ())),
            preferred_element_type=jnp.float32)
        res += b2_ref[pl.ds(k, 1), :]

        @pl.when(k >= _RO)
        def _wait_out_slot():
            out_copy(k - _RO, so).wait()
        outbuf[so] = res
        out_copy(k, so).start()

        @pl.when(k + _RB < _NFULL)
        def _issue_next():
            b_copy(k + _RB, s).start()
        return carry

    jax.lax.fori_loop(0, _NFULL, b_body, None)

    pltpu.make_async_copy(w2_hbm.at[pl.ds(_NFULL * _CK, _TAIL), :],
                          w2tail, sem_w2t).wait()
    wbt = w2tail[...].astype(jnp.bfloat16)
    rest = jax.lax.dot_general(h_ref[...], wbt, (((1,), (1,)), ((), ())),
                               preferred_element_type=jnp.float32)
    rest += b2_ref[pl.ds(_NFULL, 1), :][:, :_TAIL]
    outtail[...] = rest
    pltpu.make_async_copy(outtail,
                          out_hbm.at[:, pl.ds(_NFULL * _CK, _TAIL)],
                          sem_w2t).start()

    for j in range(_NFULL - _RO, _NFULL):
        out_copy(j, j % _RO).wait()
    pltpu.make_async_copy(outtail,
                          out_hbm.at[:, pl.ds(_NFULL * _CK, _TAIL)],
                          sem_w2t).wait()


def kernel(query, W_q, b_q, W1, b1, W2, b2, top_k):
    del top_k
    batch, vocab = query.shape
    embed = W_q.shape[0]
    hidden = W1.shape[0]

    ce = jax.random.normal(jax.random.key(42), (batch, embed),
                           dtype=jnp.float32)
    nrows = _NFULL + 1
    b2_rows = jnp.pad(b2, (0, nrows * _CK - vocab)).reshape(nrows, _CK)

    hbm = pl.BlockSpec(memory_space=pltpu.MemorySpace.HBM)
    vmem = pl.BlockSpec(memory_space=pltpu.MemorySpace.VMEM)

    out = pl.pallas_call(
        _rag_kernel,
        in_specs=[hbm, hbm, vmem, vmem, vmem, vmem, vmem, hbm],
        out_specs=hbm,
        out_shape=jax.ShapeDtypeStruct((batch, vocab), jnp.float32),
        scratch_shapes=[
            pltpu.VMEM((_RA, batch, _CK), jnp.float32),
            pltpu.VMEM((_RA, embed, _CK), jnp.float32),
            pltpu.VMEM((batch, _TAIL), jnp.float32),
            pltpu.VMEM((embed, _TAIL), jnp.float32),
            pltpu.VMEM((_RB, _CK, hidden), jnp.float32),
            pltpu.VMEM((_TAIL, hidden), jnp.float32),
            pltpu.VMEM((_RO, batch, _CK), jnp.float32),
            pltpu.VMEM((batch, _TAIL), jnp.float32),
            pltpu.VMEM((batch, embed), jnp.float32),
            pltpu.VMEM((batch, hidden), jnp.bfloat16),
            pltpu.SemaphoreType.DMA((_RA,)),
            pltpu.SemaphoreType.DMA((_RA,)),
            pltpu.SemaphoreType.DMA,
            pltpu.SemaphoreType.DMA,
            pltpu.SemaphoreType.DMA((_RB,)),
            pltpu.SemaphoreType.DMA,
            pltpu.SemaphoreType.DMA((_RO,)),
        ],
        compiler_params=pltpu.CompilerParams(
            vmem_limit_bytes=100 * 1024 * 1024),
    )(query, W_q, b_q.reshape(1, embed), W1, b1.reshape(1, hidden), ce,
      b2_rows, W2)

    return out
